# baseline (speedup 1.0000x reference)
import functools
import math

import jax
import jax.numpy as jnp
from jax import lax
from jax.experimental import pallas as pl
from jax.experimental.pallas import tpu as pltpu
from jax.experimental.pallas import tpu_sc as plsc

N = 10000
E = 320000
H = 128
G = 50
F = 128
CUTOFF = 10.0
_LN2 = math.log(2.0)

NC = 2
NS = 16
NW = NC * NS
EW = E // NW
B = 40
NCH = EW // B
NP = 10240
ROWS_PER_TILE = NP // NS
GRP = 25
NGRP = NCH // GRP


def _ssp(v):
    return jnp.maximum(v, 0.0) + jnp.log(1.0 + jnp.exp(-jnp.abs(v))) - _LN2


_BV = 400


def _prep_body(el_ref, x_ref, w_ref, c_ref, h_ref):
    el = el_ref[...]
    c = 0.5 * (jnp.cos(el * (math.pi / CUTOFF)) + 1.0)
    c_ref[...] = (c * (el <= CUTOFF).astype(jnp.float32)
                  * (el >= 0.0).astype(jnp.float32))
    h_ref[...] = jnp.dot(x_ref[...], w_ref[...],
                         preferred_element_type=jnp.float32)


def _prep(edge_length, x, w1t):
    return pl.pallas_call(
        _prep_body,
        grid=(5,),
        in_specs=[
            pl.BlockSpec((_BV, 160), lambda i: (i, 0)),
            pl.BlockSpec((_BN, H), lambda i: (i, 0)),
            pl.BlockSpec((H, F), lambda i: (0, 0)),
        ],
        out_specs=[
            pl.BlockSpec((_BV, 160), lambda i: (i, 0)),
            pl.BlockSpec((_BN, F), lambda i: (i, 0)),
        ],
        out_shape=[
            jax.ShapeDtypeStruct((E // 160, 160), jnp.float32),
            jax.ShapeDtypeStruct((N, F), jnp.float32),
        ],
    )(edge_length.reshape(E // 160, 160), x, w1t)


_BE = 2000
_BN = 2000


def _edge_mlp_body(ea_ref, c_ref, w1t_ref, b1_ref, w2t_ref, b2_ref, wf_ref):
    hid = jnp.dot(ea_ref[...], w1t_ref[...], preferred_element_type=jnp.float32)
    hid = _ssp(hid + b1_ref[...])
    wf = jnp.dot(hid, w2t_ref[...], preferred_element_type=jnp.float32)
    wf_ref[...] = (wf + b2_ref[...]) * c_ref[...]


def _edge_mlp(edge_attr, cenv, w1t, b1, w2t, b2):
    nb = E // _BE
    return pl.pallas_call(
        _edge_mlp_body,
        grid=(nb,),
        in_specs=[
            pl.BlockSpec((_BE, G), lambda i: (i, 0)),
            pl.BlockSpec((_BE, 1), lambda i: (i, 0)),
            pl.BlockSpec((G, F), lambda i: (0, 0)),
            pl.BlockSpec((1, F), lambda i: (0, 0)),
            pl.BlockSpec((F, F), lambda i: (0, 0)),
            pl.BlockSpec((1, F), lambda i: (0, 0)),
        ],
        out_specs=pl.BlockSpec((_BE, F), lambda i: (i, 0)),
        out_shape=jax.ShapeDtypeStruct((E, F), jnp.float32),
    )(edge_attr, cenv.reshape(E, 1), w1t, b1.reshape(1, F),
      w2t, b2.reshape(1, F))


def _sc_message_pass(h, wf, src3, dst3):
    mesh = plsc.VectorSubcoreMesh(core_axis_name="c", subcore_axis_name="s")

    @functools.partial(
        pl.kernel,
        mesh=mesh,
        out_type=[
            jax.ShapeDtypeStruct((E, F), jnp.float32),
            jax.ShapeDtypeStruct((NC, NP, F), jnp.float32),
        ],
        scratch_types=[
            pltpu.VMEM((GRP, B), jnp.int32),
            pltpu.VMEM((GRP, B), jnp.int32),
            pltpu.VMEM((B, F), jnp.float32),
            pltpu.VMEM((B, F), jnp.float32),
            pltpu.VMEM((B, F), jnp.float32),
            pltpu.VMEM((B, F), jnp.float32),
            pltpu.VMEM((B, F), jnp.float32),
            pltpu.VMEM((B, F), jnp.float32),
            pltpu.VMEM_SHARED((NP, F), jnp.float32),
            pltpu.SemaphoreType.DMA,
            pltpu.SemaphoreType.DMA,
            pltpu.SemaphoreType.DMA,
            pltpu.SemaphoreType.DMA,
            pltpu.SemaphoreType.DMA,
            pltpu.SemaphoreType.DMA,
            pltpu.SemaphoreType.DMA,
            pltpu.SemaphoreType.DMA,
            pltpu.SemaphoreType.DMA,
            pltpu.SemaphoreType.DMA,
            pltpu.SemaphoreType.DMA,
            pltpu.SemaphoreType.DMA,
        ],
    )
    def run(h_hbm, wf_hbm, src_hbm, dst_hbm, mij_hbm, part_hbm,
            src_v, dst_v, rows0, rows1, rows2, wf0, wf1, wf2, acc,
            sg0, sg1, sg2, sw0, sw1, sw2, sm0, sm1, sm2, ss0, ss1, ss2):
        cid = lax.axis_index("c")
        sid = lax.axis_index("s")
        wid = sid * NC + cid
        buf = ((rows0, wf0, sg0, sw0, sm0, ss0),
               (rows1, wf1, sg1, sw1, sm1, ss1),
               (rows2, wf2, sg2, sw2, sm2, ss2))

        def zrow(i, c2):
            for k in range(F // 16):
                rows0[i, pl.ds(k * 16, 16)] = jnp.zeros((16,), jnp.float32)
            return c2

        lax.fori_loop(0, B, zrow, 0)
        for r in range(ROWS_PER_TILE // B):
            pltpu.sync_copy(rows0,
                            acc.at[pl.ds(sid * ROWS_PER_TILE + r * B, B)])
        plsc.subcore_barrier()

        def issue_loads(g, j, b):
            rows, wfv, sg, sw = buf[b][:4]
            base = wid * EW + (g * GRP + j) * B
            pltpu.async_copy(h_hbm.at[src_v.at[j]], rows, sg)
            pltpu.async_copy(wf_hbm.at[pl.ds(base, B)], wfv, sw)

        def wait_loads(g, j, b):
            rows, wfv, sg, sw = buf[b][:4]
            base = wid * EW + (g * GRP + j) * B
            pltpu.make_async_copy(wf_hbm.at[pl.ds(base, B)], wfv, sw).wait()
            pltpu.make_async_copy(h_hbm.at[src_v.at[j]], rows, sg).wait()

        def issue_stores(g, j, b):
            rows, _, _, _, sm, ss = buf[b]
            base = wid * EW + (g * GRP + j) * B
            pltpu.async_copy(rows, mij_hbm.at[pl.ds(base, B)], sm)
            pltpu.async_copy(rows, acc.at[dst_v.at[j]], ss, add=True)

        def wait_stores(g, j, b):
            rows, _, _, _, sm, ss = buf[b]
            base = wid * EW + (g * GRP + j) * B
            pltpu.make_async_copy(rows, mij_hbm.at[pl.ds(base, B)], sm).wait()
            pltpu.make_async_copy(rows, acc.at[dst_v.at[j]], ss).wait()

        def multiply(b):
            rows, wfv = buf[b][:2]

            @plsc.parallel_loop(0, B, unroll=2)
            def row(i):
                for k in range(F // 16):
                    sl = pl.ds(k * 16, 16)
                    rows[i, sl] = rows[i, sl] * wfv[i, sl]

        def stage(g, j, b):
            @pl.when(j + 2 < GRP)
            def _():
                @pl.when(j >= 1)
                def _():
                    wait_stores(g, j - 1, (b + 2) % 3)

                issue_loads(g, j + 2, (b + 2) % 3)

            wait_loads(g, j, b)
            multiply(b)
            issue_stores(g, j, b)

        for g in range(NGRP):
            pltpu.sync_copy(src_hbm.at[wid * NGRP + g], src_v)
            pltpu.sync_copy(dst_hbm.at[wid * NGRP + g], dst_v)
            issue_loads(g, 0, 0)
            issue_loads(g, 1, 1)

            def triple(t, carry, g=g):
                stage(g, 3 * t, 0)
                stage(g, 3 * t + 1, 1)
                stage(g, 3 * t + 2, 2)
                return carry

            lax.fori_loop(0, GRP // 3, triple, 0)
            for j in range((GRP // 3) * 3, GRP):
                stage(g, j, j % 3)
            for j in range(GRP - 3, GRP):
                wait_stores(g, j, j % 3)

        plsc.subcore_barrier()
        pltpu.sync_copy(acc.at[pl.ds(sid * ROWS_PER_TILE, ROWS_PER_TILE)],
                        part_hbm.at[cid, pl.ds(sid * ROWS_PER_TILE,
                                               ROWS_PER_TILE)])

    return run(h, wf, src3, dst3)


def _node_update_body(x_ref, p_ref, w2t_ref, b2_ref, wxt_ref, wmt_ref,
                      bl_ref, o_ref):
    m = p_ref[0] + p_ref[1]
    m = jnp.dot(m, w2t_ref[...], preferred_element_type=jnp.float32)
    m = _ssp(m + b2_ref[...])
    o = jnp.dot(x_ref[...], wxt_ref[...], preferred_element_type=jnp.float32)
    o = o + jnp.dot(m, wmt_ref[...], preferred_element_type=jnp.float32)
    o_ref[...] = o + bl_ref[...]


def _node_update(x, parts, w2t, b2, wxt, wmt, blin):
    nb = N // _BN
    return pl.pallas_call(
        _node_update_body,
        grid=(nb,),
        in_specs=[
            pl.BlockSpec((_BN, H), lambda i: (i, 0)),
            pl.BlockSpec((NC, _BN, F), lambda i: (0, i, 0)),
            pl.BlockSpec((F, H), lambda i: (0, 0)),
            pl.BlockSpec((1, H), lambda i: (0, 0)),
            pl.BlockSpec((H, H), lambda i: (0, 0)),
            pl.BlockSpec((H, H), lambda i: (0, 0)),
            pl.BlockSpec((1, H), lambda i: (0, 0)),
        ],
        out_specs=pl.BlockSpec((_BN, H), lambda i: (i, 0)),
        out_shape=jax.ShapeDtypeStruct((N, H), jnp.float32),
    )(x, parts, w2t, b2.reshape(1, H), wxt, wmt, blin.reshape(1, H))


def kernel(x, edge_index, edge_length, edge_attr,
           W1, mlp_w1, mlp_b1, mlp_w2, mlp_b2, W2, b2, Wlin, blin):
    src3 = edge_index[0].astype(jnp.int32).reshape(NW * NGRP, GRP, B)
    dst3 = edge_index[1].astype(jnp.int32).reshape(NW * NGRP, GRP, B)
    cenv, h = _prep(edge_length, x, W1.T)
    wfilt = _edge_mlp(edge_attr, cenv, mlp_w1.T, mlp_b1,
                      mlp_w2.T, mlp_b2)
    m_ij, parts = _sc_message_pass(h, wfilt, src3, dst3)
    out = _node_update(x, parts, W2.T, b2, Wlin[:, :H].T, Wlin[:, H:].T, blin)
    return (out, m_ij)


# speedup vs baseline: 1.1505x; 1.1505x over previous
import functools
import math

import jax
import jax.numpy as jnp
from jax import lax
from jax.experimental import pallas as pl
from jax.experimental.pallas import tpu as pltpu
from jax.experimental.pallas import tpu_sc as plsc

N = 10000
E = 320000
H = 128
G = 50
F = 128
CUTOFF = 10.0
_LN2 = math.log(2.0)

NC = 2
NS = 16
NW = NC * NS
EW = E // NW
B = 40
NCH = EW // B
NP = 10240
ROWS_PER_TILE = NP // NS
GRP = 50
NGRP = NCH // GRP


def _ssp(v):
    return jnp.maximum(v, 0.0) + jnp.log(1.0 + jnp.exp(-jnp.abs(v))) - _LN2


def _envelope_body(el_ref, c_ref):
    el = el_ref[...]
    c = 0.5 * (jnp.cos(el * (math.pi / CUTOFF)) + 1.0)
    c_ref[...] = (c * (el <= CUTOFF).astype(jnp.float32)
                  * (el >= 0.0).astype(jnp.float32))


def _envelope(edge_length):
    return pl.pallas_call(
        _envelope_body,
        out_shape=jax.ShapeDtypeStruct((E // 128, 128), jnp.float32),
    )(edge_length.reshape(E // 128, 128))


def _hproj_body(x_ref, w_ref, o_ref):
    o_ref[...] = jnp.dot(x_ref[...], w_ref[...],
                         preferred_element_type=jnp.float32)


def _hproj(x, w1t):
    nb = N // _BN
    return pl.pallas_call(
        _hproj_body,
        grid=(nb,),
        in_specs=[
            pl.BlockSpec((_BN, H), lambda i: (i, 0)),
            pl.BlockSpec((H, F), lambda i: (0, 0)),
        ],
        out_specs=pl.BlockSpec((_BN, F), lambda i: (i, 0)),
        out_shape=jax.ShapeDtypeStruct((N, F), jnp.float32),
    )(x, w1t)


_BE = 2000
_BN = 2000


def _edge_mlp_body(ea_ref, c_ref, w1t_ref, b1_ref, w2t_ref, b2_ref, wf_ref):
    hid = jnp.dot(ea_ref[...], w1t_ref[...], preferred_element_type=jnp.float32)
    hid = _ssp(hid + b1_ref[...])
    wf = jnp.dot(hid, w2t_ref[...], preferred_element_type=jnp.float32)
    wf_ref[...] = (wf + b2_ref[...]) * c_ref[...]


def _edge_mlp(edge_attr, cenv, w1t, b1, w2t, b2):
    nb = E // _BE
    return pl.pallas_call(
        _edge_mlp_body,
        grid=(nb,),
        in_specs=[
            pl.BlockSpec((_BE, G), lambda i: (i, 0)),
            pl.BlockSpec((_BE, 1), lambda i: (i, 0)),
            pl.BlockSpec((G, F), lambda i: (0, 0)),
            pl.BlockSpec((1, F), lambda i: (0, 0)),
            pl.BlockSpec((F, F), lambda i: (0, 0)),
            pl.BlockSpec((1, F), lambda i: (0, 0)),
        ],
        out_specs=pl.BlockSpec((_BE, F), lambda i: (i, 0)),
        out_shape=jax.ShapeDtypeStruct((E, F), jnp.float32),
    )(edge_attr, cenv.reshape(E, 1), w1t, b1.reshape(1, F),
      w2t, b2.reshape(1, F))


def _sc_message_pass(h, wf, src3, dst3):
    mesh = plsc.VectorSubcoreMesh(core_axis_name="c", subcore_axis_name="s")

    @functools.partial(
        pl.kernel,
        mesh=mesh,
        out_type=[
            jax.ShapeDtypeStruct((E, F), jnp.float32),
            jax.ShapeDtypeStruct((NC, NP, F), jnp.float32),
        ],
        scratch_types=[
            pltpu.VMEM((GRP, B), jnp.int32),
            pltpu.VMEM((GRP, B), jnp.int32),
            pltpu.VMEM((B, F), jnp.float32),
            pltpu.VMEM((B, F), jnp.float32),
            pltpu.VMEM((B, F), jnp.float32),
            pltpu.VMEM((B, F), jnp.float32),
            pltpu.VMEM((B, F), jnp.float32),
            pltpu.VMEM((B, F), jnp.float32),
            pltpu.VMEM_SHARED((NP, F), jnp.float32),
            pltpu.SemaphoreType.DMA,
            pltpu.SemaphoreType.DMA,
            pltpu.SemaphoreType.DMA,
            pltpu.SemaphoreType.DMA,
            pltpu.SemaphoreType.DMA,
            pltpu.SemaphoreType.DMA,
            pltpu.SemaphoreType.DMA,
            pltpu.SemaphoreType.DMA,
            pltpu.SemaphoreType.DMA,
            pltpu.SemaphoreType.DMA,
            pltpu.SemaphoreType.DMA,
            pltpu.SemaphoreType.DMA,
        ],
    )
    def run(h_hbm, wf_hbm, src_hbm, dst_hbm, mij_hbm, part_hbm,
            src_v, dst_v, rows0, rows1, rows2, wf0, wf1, wf2, acc,
            sg0, sg1, sg2, sw0, sw1, sw2, sm0, sm1, sm2, ss0, ss1, ss2):
        cid = lax.axis_index("c")
        sid = lax.axis_index("s")
        wid = sid * NC + cid
        buf = ((rows0, wf0, sg0, sw0, sm0, ss0),
               (rows1, wf1, sg1, sw1, sm1, ss1),
               (rows2, wf2, sg2, sw2, sm2, ss2))

        def zrow(i, c2):
            for k in range(F // 16):
                rows0[i, pl.ds(k * 16, 16)] = jnp.zeros((16,), jnp.float32)
            return c2

        lax.fori_loop(0, B, zrow, 0)
        for r in range(ROWS_PER_TILE // B):
            pltpu.sync_copy(rows0,
                            acc.at[pl.ds(sid * ROWS_PER_TILE + r * B, B)])
        plsc.subcore_barrier()

        def issue_loads(g, j, b):
            rows, wfv, sg, sw = buf[b][:4]
            base = wid * EW + (g * GRP + j) * B
            pltpu.async_copy(h_hbm.at[src_v.at[j]], rows, sg)
            pltpu.async_copy(wf_hbm.at[pl.ds(base, B)], wfv, sw)

        def wait_loads(g, j, b):
            rows, wfv, sg, sw = buf[b][:4]
            base = wid * EW + (g * GRP + j) * B
            pltpu.make_async_copy(wf_hbm.at[pl.ds(base, B)], wfv, sw).wait()
            pltpu.make_async_copy(h_hbm.at[src_v.at[j]], rows, sg).wait()

        def issue_stores(g, j, b):
            rows, _, _, _, sm, ss = buf[b]
            base = wid * EW + (g * GRP + j) * B
            pltpu.async_copy(rows, mij_hbm.at[pl.ds(base, B)], sm)
            pltpu.async_copy(rows, acc.at[dst_v.at[j]], ss, add=True)

        def wait_stores(g, j, b):
            rows, _, _, _, sm, ss = buf[b]
            base = wid * EW + (g * GRP + j) * B
            pltpu.make_async_copy(rows, mij_hbm.at[pl.ds(base, B)], sm).wait()
            pltpu.make_async_copy(rows, acc.at[dst_v.at[j]], ss).wait()

        def multiply(b):
            rows, wfv = buf[b][:2]

            @plsc.parallel_loop(0, B, unroll=4)
            def row(i):
                for k in range(F // 16):
                    sl = pl.ds(k * 16, 16)
                    rows[i, sl] = rows[i, sl] * wfv[i, sl]

        def stage(g, j, b):
            @pl.when(j + 2 < GRP)
            def _():
                @pl.when(j >= 1)
                def _():
                    wait_stores(g, j - 1, (b + 2) % 3)

                issue_loads(g, j + 2, (b + 2) % 3)

            wait_loads(g, j, b)
            multiply(b)
            issue_stores(g, j, b)

        for g in range(NGRP):
            pltpu.sync_copy(src_hbm.at[wid * NGRP + g], src_v)
            pltpu.sync_copy(dst_hbm.at[wid * NGRP + g], dst_v)
            issue_loads(g, 0, 0)
            issue_loads(g, 1, 1)

            def triple(t, carry, g=g):
                stage(g, 3 * t, 0)
                stage(g, 3 * t + 1, 1)
                stage(g, 3 * t + 2, 2)
                return carry

            lax.fori_loop(0, GRP // 3, triple, 0)
            for j in range((GRP // 3) * 3, GRP):
                stage(g, j, j % 3)
            for j in range(GRP - 3, GRP):
                wait_stores(g, j, j % 3)

        plsc.subcore_barrier()
        pltpu.sync_copy(acc.at[pl.ds(sid * ROWS_PER_TILE, ROWS_PER_TILE)],
                        part_hbm.at[cid, pl.ds(sid * ROWS_PER_TILE,
                                               ROWS_PER_TILE)])

    return run(h, wf, src3, dst3)


def _node_update_body(x_ref, p_ref, w2t_ref, b2_ref, wxt_ref, wmt_ref,
                      bl_ref, o_ref):
    m = p_ref[0] + p_ref[1]
    m = jnp.dot(m, w2t_ref[...], preferred_element_type=jnp.float32)
    m = _ssp(m + b2_ref[...])
    o = jnp.dot(x_ref[...], wxt_ref[...], preferred_element_type=jnp.float32)
    o = o + jnp.dot(m, wmt_ref[...], preferred_element_type=jnp.float32)
    o_ref[...] = o + bl_ref[...]


def _node_update(x, parts, w2t, b2, wxt, wmt, blin):
    nb = N // _BN
    return pl.pallas_call(
        _node_update_body,
        grid=(nb,),
        in_specs=[
            pl.BlockSpec((_BN, H), lambda i: (i, 0)),
            pl.BlockSpec((NC, _BN, F), lambda i: (0, i, 0)),
            pl.BlockSpec((F, H), lambda i: (0, 0)),
            pl.BlockSpec((1, H), lambda i: (0, 0)),
            pl.BlockSpec((H, H), lambda i: (0, 0)),
            pl.BlockSpec((H, H), lambda i: (0, 0)),
            pl.BlockSpec((1, H), lambda i: (0, 0)),
        ],
        out_specs=pl.BlockSpec((_BN, H), lambda i: (i, 0)),
        out_shape=jax.ShapeDtypeStruct((N, H), jnp.float32),
    )(x, parts, w2t, b2.reshape(1, H), wxt, wmt, blin.reshape(1, H))


def kernel(x, edge_index, edge_length, edge_attr,
           W1, mlp_w1, mlp_b1, mlp_w2, mlp_b2, W2, b2, Wlin, blin):
    src3 = edge_index[0].astype(jnp.int32).reshape(NW * NGRP, GRP, B)
    dst3 = edge_index[1].astype(jnp.int32).reshape(NW * NGRP, GRP, B)
    cenv = _envelope(edge_length)
    wfilt = _edge_mlp(edge_attr, cenv, mlp_w1.T, mlp_b1,
                      mlp_w2.T, mlp_b2)
    h = _hproj(x, W1.T)
    m_ij, parts = _sc_message_pass(h, wfilt, src3, dst3)
    out = _node_update(x, parts, W2.T, b2, Wlin[:, :H].T, Wlin[:, H:].T, blin)
    return (out, m_ij)
